# baseline (speedup 1.0000x reference)
import functools

import jax
import jax.numpy as jnp
from jax import lax
from jax.experimental import pallas as pl
from jax.experimental.pallas import tpu as pltpu
from jax.experimental.pallas import tpu_sc as plsc

N_GLOBAL = 2048
NUM_CLIENTS = 8
L_LOCAL = 1024
TAU_C = 0.5
GAMMA = 1.0
TAU_S = 0.5

ROW_TILE = 256
_LANES = 16


_TOTAL = NUM_CLIENTS * N_GLOBAL * N_GLOBAL
_NUPD = NUM_CLIENTS * L_LOCAL * L_LOCAL


def _sorted_updates(client_adj_list, idx32):
    coff = (jnp.arange(NUM_CLIENTS, dtype=jnp.int32) * (N_GLOBAL * N_GLOBAL))[:, None, None]
    keys = (coff + idx32[:, :, None] * N_GLOBAL + idx32[:, None, :]).reshape(-1)
    vals = client_adj_list.reshape(-1)
    return lax.sort((keys, vals), dimension=0, num_keys=1, is_stable=False)


def _scatter_winners(sk, sv):
    winner = jnp.concatenate([sk[:-1] != sk[1:], jnp.ones((1,), jnp.bool_)])
    sentinel = _TOTAL + lax.iota(jnp.int32, _NUPD)
    tgt = jnp.where(winner, sk, sentinel)
    GA_flat = jnp.zeros((_TOTAL,), jnp.float32).at[tgt].set(sv, unique_indices=True)
    return GA_flat.reshape(NUM_CLIENTS, N_GLOBAL, N_GLOBAL)


def _presence_kernel(idx_hbm, out_hbm, idx_v, pres_v):
    w = lax.axis_index("s") * 2 + lax.axis_index("c")

    @pl.when(w < NUM_CLIENTS)
    def _():
        pltpu.sync_copy(idx_hbm.at[w], idx_v)

        def zero_body(i, carry):
            pres_v[pl.ds(i * _LANES, _LANES)] = jnp.zeros((_LANES,), jnp.float32)
            return carry

        lax.fori_loop(0, N_GLOBAL // _LANES, zero_body, 0)

        ones = jnp.full((_LANES,), 1.0, jnp.float32)

        def scat_body(i, carry):
            v = idx_v[pl.ds(i * _LANES, _LANES)]
            plsc.store_scatter(pres_v, [v], ones)
            return carry

        lax.fori_loop(0, L_LOCAL // _LANES, scat_body, 0)
        pltpu.sync_copy(pres_v, out_hbm.at[w])


@functools.partial(
    pl.kernel,
    out_type=jax.ShapeDtypeStruct((NUM_CLIENTS, N_GLOBAL), jnp.float32),
    mesh=plsc.VectorSubcoreMesh(core_axis_name="c", subcore_axis_name="s"),
    compiler_params=pltpu.CompilerParams(needs_layout_passes=False),
    scratch_types=[
        pltpu.VMEM((L_LOCAL,), jnp.int32),
        pltpu.VMEM((N_GLOBAL,), jnp.float32),
    ],
)
def _presence(idx_hbm, out_hbm, idx_v, pres_v):
    _presence_kernel(idx_hbm, out_hbm, idx_v, pres_v)


def _stats_kernel(ga_ref, pres_ref, presT_ref, out_ref, n_ref, s_ref, ss_ref, cnt_ref):
    c = pl.program_id(1)

    v = ga_ref[0]
    lane_oh = (lax.broadcasted_iota(jnp.int32, (1, NUM_CLIENTS), 1) == c).astype(jnp.float32)
    sub_oh = (lax.broadcasted_iota(jnp.int32, (NUM_CLIENTS, 1), 0) == c).astype(jnp.float32)
    rowp = jnp.sum(presT_ref[...] * lane_oh, axis=1, keepdims=True)
    colp = jnp.sum(pres_ref[...] * sub_oh, axis=0, keepdims=True)
    m = rowp * colp
    sv = v * m
    above = jnp.where(v > TAU_C, m, 0.0)

    @pl.when(c == 0)
    def _():
        n_ref[...] = m
        s_ref[...] = sv
        ss_ref[...] = sv * v
        cnt_ref[...] = above

    @pl.when(c > 0)
    def _():
        n_ref[...] += m
        s_ref[...] += sv
        ss_ref[...] += sv * v
        cnt_ref[...] += above

    @pl.when(c == NUM_CLIENTS - 1)
    def _():
        n = n_ref[...]
        nc = jnp.maximum(n, 1e-05)
        mean = s_ref[...] / nc
        var = ss_ref[...] / nc - mean * mean
        cfrac = cnt_ref[...] / nc
        score = cfrac * jnp.exp(-GAMMA * var)
        keep = jnp.logical_and(score > TAU_S, n > 1e-05)
        out_ref[...] = jnp.where(keep, mean, 0.0)


def _fused_stats(GA, pres, presT):
    grid = (N_GLOBAL // ROW_TILE, NUM_CLIENTS)
    return pl.pallas_call(
        _stats_kernel,
        grid=grid,
        in_specs=[
            pl.BlockSpec((1, ROW_TILE, N_GLOBAL), lambda t, c: (c, t, 0)),
            pl.BlockSpec((NUM_CLIENTS, N_GLOBAL), lambda t, c: (0, 0)),
            pl.BlockSpec((ROW_TILE, NUM_CLIENTS), lambda t, c: (t, 0)),
        ],
        out_specs=pl.BlockSpec((ROW_TILE, N_GLOBAL), lambda t, c: (t, 0)),
        out_shape=jax.ShapeDtypeStruct((N_GLOBAL, N_GLOBAL), jnp.float32),
        scratch_shapes=[pltpu.VMEM((ROW_TILE, N_GLOBAL), jnp.float32)] * 4,
        compiler_params=pltpu.CompilerParams(
            dimension_semantics=("parallel", "arbitrary"),
        ),
    )(GA, pres, presT)


def kernel(client_adj_list, client_node_indices):
    idx32 = client_node_indices.astype(jnp.int32)
    sk, sv = _sorted_updates(client_adj_list, idx32)
    GA = _scatter_winners(sk, sv)
    pres = _presence(idx32)
    presT = pres.T
    return _fused_stats(GA, pres, presT)


# speedup vs baseline: 4.9957x; 4.9957x over previous
import functools

import jax
import jax.numpy as jnp
from jax import lax
from jax.experimental import pallas as pl
from jax.experimental.pallas import tpu as pltpu
from jax.experimental.pallas import tpu_sc as plsc

N_GLOBAL = 2048
NUM_CLIENTS = 8
L_LOCAL = 1024
TAU_C = 0.5
GAMMA = 1.0
TAU_S = 0.5

ROW_TILE = 256
_LANES = 16
_NUPD = NUM_CLIENTS * L_LOCAL * L_LOCAL


def _sorted_updates(client_adj_list, idx32):
    coff = (jnp.arange(NUM_CLIENTS, dtype=jnp.int32) * (N_GLOBAL * N_GLOBAL))[:, None, None]
    keys = (coff + idx32[:, :, None] * N_GLOBAL + idx32[:, None, :]).reshape(-1)
    vals = client_adj_list.reshape(-1)
    return lax.sort((keys, vals), dimension=0, num_keys=1, is_stable=False)


def _index_stats_body(idx_hbm, a3_out, s_out, pres_out, idx_v, cnt_v, a3_v, s_v, pres_v):
    w = lax.axis_index("s") * 2 + lax.axis_index("c")

    @pl.when(w < NUM_CLIENTS)
    def _():
        pltpu.sync_copy(idx_hbm.at[w], idx_v)

        def zero_body(i, carry):
            cnt_v[pl.ds(i * _LANES, _LANES)] = jnp.zeros((_LANES,), jnp.int32)
            return carry

        lax.fori_loop(0, N_GLOBAL // _LANES, zero_body, 0)

        ones = jnp.full((_LANES,), 1, jnp.int32)

        def scat_body(i, carry):
            v = idx_v[pl.ds(i * _LANES, _LANES)]
            plsc.addupdate_scatter(cnt_v, [v], ones)
            return carry

        lax.fori_loop(0, L_LOCAL // _LANES, scat_body, 0)

        def scan_body(i, carry):
            vec = cnt_v[pl.ds(i * _LANES, _LANES)]
            inc = plsc.cumsum(vec) + carry
            a3_v[pl.ds(i * _LANES, _LANES)] = inc
            s_v[pl.ds(i * _LANES, _LANES)] = inc - vec
            pres_v[pl.ds(i * _LANES, _LANES)] = jnp.where(vec > 0, 1.0, 0.0)
            return jnp.max(inc)

        lax.fori_loop(0, N_GLOBAL // _LANES, scan_body, jnp.int32(0))
        pltpu.sync_copy(a3_v, a3_out.at[w])
        pltpu.sync_copy(s_v, s_out.at[w])
        pltpu.sync_copy(pres_v, pres_out.at[w])


@functools.partial(
    pl.kernel,
    out_type=(
        jax.ShapeDtypeStruct((NUM_CLIENTS, N_GLOBAL), jnp.int32),
        jax.ShapeDtypeStruct((NUM_CLIENTS, N_GLOBAL), jnp.int32),
        jax.ShapeDtypeStruct((NUM_CLIENTS, N_GLOBAL), jnp.float32),
    ),
    mesh=plsc.VectorSubcoreMesh(core_axis_name="c", subcore_axis_name="s"),
    compiler_params=pltpu.CompilerParams(needs_layout_passes=False),
    scratch_types=[
        pltpu.VMEM((L_LOCAL,), jnp.int32),
        pltpu.VMEM((N_GLOBAL,), jnp.int32),
        pltpu.VMEM((N_GLOBAL,), jnp.int32),
        pltpu.VMEM((N_GLOBAL,), jnp.int32),
        pltpu.VMEM((N_GLOBAL,), jnp.float32),
    ],
)
def _index_stats(idx_hbm, a3_out, s_out, pres_out, idx_v, cnt_v, a3_v, s_v, pres_v):
    _index_stats_body(idx_hbm, a3_out, s_out, pres_out, idx_v, cnt_v, a3_v, s_v, pres_v)


def _lane_scalar(vmem_ref, g):
    vec = vmem_ref[pl.ds((g // _LANES) * _LANES, _LANES)]
    lane = g % _LANES
    sel = jnp.where(lax.iota(jnp.int32, _LANES) == lane, vec, 0)
    return jnp.max(sel)


def _expand_body(sv_hbm, a3_hbm, s_hbm, out_hbm, a3_v, s_v, pos_v, row_v, zrow_v, sem):
    w = lax.axis_index("s") * 2 + lax.axis_index("c")
    cli = w // 4
    q = w % 4
    pltpu.sync_copy(a3_hbm.at[cli], a3_v)
    pltpu.sync_copy(s_hbm.at[cli], s_v)

    def zero_body(i, carry):
        zrow_v[pl.ds(i * _LANES, _LANES)] = jnp.zeros((_LANES,), jnp.float32)
        return carry

    lax.fori_loop(0, N_GLOBAL // _LANES, zero_body, 0)

    base_c = cli * (L_LOCAL * L_LOCAL)

    def row_body(r, carry):
        g1 = q * 512 + r
        s_scal = _lane_scalar(s_v, g1)
        a3_scal = _lane_scalar(a3_v, g1)
        d_scal = a3_scal - s_scal
        base = base_c + s_scal * L_LOCAL - 1
        fr = cli * N_GLOBAL + g1

        @pl.when(d_scal > 0)
        def _():
            def pos_body(j, c2):
                a3c = a3_v[pl.ds(j * _LANES, _LANES)]
                pos = jnp.maximum(base + d_scal * a3c, 0)
                pos_v[pl.ds(j * _LANES, _LANES)] = pos
                return c2

            lax.fori_loop(0, N_GLOBAL // _LANES, pos_body, 0)
            pltpu.async_copy(sv_hbm.at[pos_v], row_v, sem).wait()
            pltpu.sync_copy(row_v, out_hbm.at[fr])

        @pl.when(d_scal == 0)
        def _():
            pltpu.sync_copy(zrow_v, out_hbm.at[fr])

        return carry

    lax.fori_loop(0, 512, row_body, 0)


@functools.partial(
    pl.kernel,
    out_type=jax.ShapeDtypeStruct((NUM_CLIENTS * N_GLOBAL, N_GLOBAL), jnp.float32),
    mesh=plsc.VectorSubcoreMesh(core_axis_name="c", subcore_axis_name="s"),
    compiler_params=pltpu.CompilerParams(needs_layout_passes=False),
    scratch_types=[
        pltpu.VMEM((N_GLOBAL,), jnp.int32),
        pltpu.VMEM((N_GLOBAL,), jnp.int32),
        pltpu.VMEM((N_GLOBAL,), jnp.int32),
        pltpu.VMEM((N_GLOBAL,), jnp.float32),
        pltpu.VMEM((N_GLOBAL,), jnp.float32),
        pltpu.SemaphoreType.DMA,
    ],
)
def _expand(sv_hbm, a3_hbm, s_hbm, out_hbm, a3_v, s_v, pos_v, row_v, zrow_v, sem):
    _expand_body(sv_hbm, a3_hbm, s_hbm, out_hbm, a3_v, s_v, pos_v, row_v, zrow_v, sem)


def _stats_kernel(ga_ref, pres_ref, presT_ref, out_ref, n_ref, s_ref, ss_ref, cnt_ref):
    c = pl.program_id(1)

    v = ga_ref[0]
    lane_oh = (lax.broadcasted_iota(jnp.int32, (1, NUM_CLIENTS), 1) == c).astype(jnp.float32)
    sub_oh = (lax.broadcasted_iota(jnp.int32, (NUM_CLIENTS, 1), 0) == c).astype(jnp.float32)
    rowp = jnp.sum(presT_ref[...] * lane_oh, axis=1, keepdims=True)
    colp = jnp.sum(pres_ref[...] * sub_oh, axis=0, keepdims=True)
    m = rowp * colp
    sv = v * m
    above = jnp.where(v > TAU_C, m, 0.0)

    @pl.when(c == 0)
    def _():
        n_ref[...] = m
        s_ref[...] = sv
        ss_ref[...] = sv * v
        cnt_ref[...] = above

    @pl.when(c > 0)
    def _():
        n_ref[...] += m
        s_ref[...] += sv
        ss_ref[...] += sv * v
        cnt_ref[...] += above

    @pl.when(c == NUM_CLIENTS - 1)
    def _():
        n = n_ref[...]
        nc = jnp.maximum(n, 1e-05)
        mean = s_ref[...] / nc
        var = ss_ref[...] / nc - mean * mean
        cfrac = cnt_ref[...] / nc
        score = cfrac * jnp.exp(-GAMMA * var)
        keep = jnp.logical_and(score > TAU_S, n > 1e-05)
        out_ref[...] = jnp.where(keep, mean, 0.0)


def _fused_stats(GA, pres, presT):
    grid = (N_GLOBAL // ROW_TILE, NUM_CLIENTS)
    return pl.pallas_call(
        _stats_kernel,
        grid=grid,
        in_specs=[
            pl.BlockSpec((1, ROW_TILE, N_GLOBAL), lambda t, c: (c, t, 0)),
            pl.BlockSpec((NUM_CLIENTS, N_GLOBAL), lambda t, c: (0, 0)),
            pl.BlockSpec((ROW_TILE, NUM_CLIENTS), lambda t, c: (t, 0)),
        ],
        out_specs=pl.BlockSpec((ROW_TILE, N_GLOBAL), lambda t, c: (t, 0)),
        out_shape=jax.ShapeDtypeStruct((N_GLOBAL, N_GLOBAL), jnp.float32),
        scratch_shapes=[pltpu.VMEM((ROW_TILE, N_GLOBAL), jnp.float32)] * 4,
        compiler_params=pltpu.CompilerParams(
            dimension_semantics=("parallel", "arbitrary"),
        ),
    )(GA, pres, presT)


def kernel(client_adj_list, client_node_indices):
    idx32 = client_node_indices.astype(jnp.int32)
    _, sv = _sorted_updates(client_adj_list, idx32)
    a3, s, pres = _index_stats(idx32)
    GA = _expand(sv, a3, s).reshape(NUM_CLIENTS, N_GLOBAL, N_GLOBAL)
    return _fused_stats(GA, pres, pres.T)
